# baseline (speedup 1.0000x reference)
import functools

import jax
import jax.numpy as jnp
from jax import lax
from jax.experimental import pallas as pl
from jax.experimental.pallas import tpu as pltpu
from jax.experimental.pallas import tpu_sc as plsc

BATCH = 16384
EMB = 32
NROWS = 1000000

_info = plsc.get_sparse_core_info()
NC, NS, L = _info.num_cores, _info.num_subcores, _info.num_lanes
NW = NC * NS
B_PER_W = BATCH // NW
N_GROUP = B_PER_W // L
N_ELEM = B_PER_W * EMB
CHUNK = 128
N_CHUNK = N_ELEM // CHUNK
FIRE = 8

_mesh = plsc.VectorSubcoreMesh(core_axis_name="c", subcore_axis_name="s")


@functools.partial(
    pl.kernel,
    mesh=_mesh,
    out_type=jax.ShapeDtypeStruct((BATCH,), jnp.float32),
    scratch_types=[
        pltpu.VMEM((B_PER_W,), jnp.int32),
        pltpu.VMEM((B_PER_W,), jnp.int32),
        pltpu.VMEM((N_ELEM,), jnp.int32),
        pltpu.VMEM((N_ELEM,), jnp.int32),
        pltpu.VMEM((N_ELEM,), jnp.float32),
        pltpu.VMEM((N_ELEM,), jnp.float32),
        pltpu.VMEM((B_PER_W,), jnp.float32),
        pltpu.SemaphoreType.DMA,
    ],
    compiler_params=pltpu.CompilerParams(
        needs_layout_passes=False, use_tc_tiling_on_sc=False),
)
def _mf_sc(u_hbm, v_hbm, ue1, ie1, out_hbm,
           iu, iv, offu, offv, ubuf, vbuf, out_v, sem):
    wid = lax.axis_index("s") * NC + lax.axis_index("c")
    base = wid * B_PER_W

    pltpu.sync_copy(u_hbm.at[pl.ds(base, B_PER_W)], iu)
    pltpu.sync_copy(v_hbm.at[pl.ds(base, B_PER_W)], iv)

    def expand(g, carry):
        iu_vec = iu[pl.ds(g * L, L)]
        iv_vec = iv[pl.ds(g * L, L)]
        for d in range(EMB):
            s = d * B_PER_W + g * L
            offu[pl.ds(s, L)] = iu_vec + d * NROWS
            offv[pl.ds(s, L)] = iv_vec + d * NROWS
        return carry

    lax.fori_loop(0, N_GROUP, expand, 0)

    for c0 in range(0, N_CHUNK, FIRE):
        copies = []
        for c in range(c0, c0 + FIRE):
            s = c * CHUNK
            copies.append(pltpu.async_copy(
                ue1.at[offu.at[pl.ds(s, CHUNK)]],
                ubuf.at[pl.ds(s, CHUNK)], sem))
            copies.append(pltpu.async_copy(
                ie1.at[offv.at[pl.ds(s, CHUNK)]],
                vbuf.at[pl.ds(s, CHUNK)], sem))
        for cp in copies:
            cp.wait()

    def dot(g, carry):
        acc = jnp.zeros((L,), jnp.float32)
        for d in range(EMB):
            s = d * B_PER_W + g * L
            acc = acc + ubuf[pl.ds(s, L)] * vbuf[pl.ds(s, L)]
        out_v[pl.ds(g * L, L)] = acc
        return carry

    lax.fori_loop(0, N_GROUP, dot, 0)

    pltpu.sync_copy(out_v, out_hbm.at[pl.ds(base, B_PER_W)])


def kernel(u, v, users_emb, items_emb):
    return _mf_sc(u.astype(jnp.int32), v.astype(jnp.int32),
                  users_emb.T.reshape(-1), items_emb.T.reshape(-1))


# speedup vs baseline: 5.4434x; 5.4434x over previous
import functools

import jax
import jax.numpy as jnp
from jax import lax
from jax.experimental import pallas as pl
from jax.experimental.pallas import tpu as pltpu
from jax.experimental.pallas import tpu_sc as plsc

BATCH = 16384
EMB = 32

_info = plsc.get_sparse_core_info()
NC, NS, L = _info.num_cores, _info.num_subcores, _info.num_lanes
NW = NC * NS
B_PER_W = BATCH // NW
N_GROUP = B_PER_W // L
N_ELEM = B_PER_W * EMB
CHUNK = 128
N_CHUNK = N_ELEM // CHUNK
FIRE = 8

_mesh = plsc.VectorSubcoreMesh(core_axis_name="c", subcore_axis_name="s")


@functools.partial(
    pl.kernel,
    mesh=_mesh,
    out_type=jax.ShapeDtypeStruct((BATCH,), jnp.float32),
    scratch_types=[
        pltpu.VMEM((B_PER_W,), jnp.int32),
        pltpu.VMEM((B_PER_W,), jnp.int32),
        pltpu.VMEM((N_ELEM,), jnp.int32),
        pltpu.VMEM((N_ELEM,), jnp.int32),
        pltpu.VMEM((N_ELEM,), jnp.float32),
        pltpu.VMEM((N_ELEM,), jnp.float32),
        pltpu.VMEM((B_PER_W,), jnp.float32),
        pltpu.SemaphoreType.DMA,
    ],
    compiler_params=pltpu.CompilerParams(
        needs_layout_passes=False, use_tc_tiling_on_sc=False),
)
def _mf_sc(u_hbm, v_hbm, ue1, ie1, out_hbm,
           iu, iv, offu, offv, ubuf, vbuf, out_v, sem):
    wid = lax.axis_index("s") * NC + lax.axis_index("c")
    base = wid * B_PER_W

    pltpu.sync_copy(u_hbm.at[pl.ds(base, B_PER_W)], iu)
    pltpu.sync_copy(v_hbm.at[pl.ds(base, B_PER_W)], iv)

    def expand(g, carry):
        iu_vec = iu[pl.ds(g * L, L)] * EMB
        iv_vec = iv[pl.ds(g * L, L)] * EMB
        for d in range(EMB):
            s = d * B_PER_W + g * L
            offu[pl.ds(s, L)] = iu_vec + d
            offv[pl.ds(s, L)] = iv_vec + d
        return carry

    lax.fori_loop(0, N_GROUP, expand, 0)

    for c0 in range(0, N_CHUNK, FIRE):
        copies = []
        for c in range(c0, c0 + FIRE):
            s = c * CHUNK
            copies.append(pltpu.async_copy(
                ue1.at[offu.at[pl.ds(s, CHUNK)]],
                ubuf.at[pl.ds(s, CHUNK)], sem))
            copies.append(pltpu.async_copy(
                ie1.at[offv.at[pl.ds(s, CHUNK)]],
                vbuf.at[pl.ds(s, CHUNK)], sem))
        for cp in copies:
            cp.wait()

    def dot(g, carry):
        acc = jnp.zeros((L,), jnp.float32)
        for d in range(EMB):
            s = d * B_PER_W + g * L
            acc = acc + ubuf[pl.ds(s, L)] * vbuf[pl.ds(s, L)]
        out_v[pl.ds(g * L, L)] = acc
        return carry

    lax.fori_loop(0, N_GROUP, dot, 0)

    pltpu.sync_copy(out_v, out_hbm.at[pl.ds(base, B_PER_W)])


def kernel(u, v, users_emb, items_emb):
    return _mf_sc(u.astype(jnp.int32), v.astype(jnp.int32),
                  users_emb.reshape(-1), items_emb.reshape(-1))
